# Initial kernel scaffold; baseline (speedup 1.0000x reference)
import jax
import jax.numpy as jnp
from jax.experimental import pallas as pl


def kernel(frustum_features, lidar_to_cam, cam_to_img, image_shape):
    raise NotImplementedError("write your pallas kernel here")


# speedup vs baseline: 3.9741x; 3.9741x over previous
import functools

import numpy as np
import jax
import jax.numpy as jnp
from jax import lax
from jax.experimental import pallas as pl
from jax.experimental.pallas import tpu as pltpu
from jax.experimental.pallas import tpu_sc as plsc

C = 64
D_BINS, H_FEAT, W_FEAT = 80, 47, 156
GX, GY, GZ = 160, 160, 16
DHW = D_BINS * H_FEAT * W_FEAT
N_VOX = GX * GY * GZ
PC_RANGE = (0.0, -25.6, -3.0, 51.2, 25.6, 1.0)
NUM_BINS = 80
DEPTH_MIN = 2.0
DEPTH_MAX = 46.8
OOB = -2.0
BIN_SIZE = 2.0 * (DEPTH_MAX - DEPTH_MIN) / (NUM_BINS * (1 + NUM_BINS))

_PC_MIN = np.array(PC_RANGE[:3], dtype=np.float32)
_PC_MAX = np.array(PC_RANGE[3:], dtype=np.float32)
_VOX_SZ = (_PC_MAX - _PC_MIN) / np.array([GX, GY, GZ], dtype=np.float32)

TBLK = 2560
NW = 32
PER_W = N_VOX // NW
NBLK = 128
NCHUNK = PER_W // NBLK


def _transpose_body(in_ref, out_ref):
    out_ref[...] = in_ref[...].T


def _relayout_features(feat2d):
    return pl.pallas_call(
        _transpose_body,
        grid=(pl.cdiv(DHW, TBLK),),
        in_specs=[pl.BlockSpec((C, TBLK), lambda i: (0, i))],
        out_specs=pl.BlockSpec((TBLK, C), lambda i: (i, 0)),
        out_shape=jax.ShapeDtypeStruct((DHW, C), jnp.float32),
    )(feat2d)


def _bf(x):
    return x.astype(jnp.bfloat16).astype(jnp.float32)


def _grid_body(par_ref, idx_ref, w_ref):
    z = pl.program_id(0)
    l2c = [par_ref[i] for i in range(12)]
    c2i = [par_ref[12 + i] for i in range(12)]
    c2i23 = par_ref[24]
    nw = par_ref[25]
    nh = par_ref[26]

    shape = (1, GY, GX)
    ixf = lax.broadcasted_iota(jnp.int32, shape, 2).astype(jnp.float32) + 0.5
    iyf = lax.broadcasted_iota(jnp.int32, shape, 1).astype(jnp.float32) + 0.5
    izf = z.astype(jnp.float32) + 0.5
    lx = _bf(ixf * _VOX_SZ[0] + _PC_MIN[0])
    ly = _bf(iyf * _VOX_SZ[1] + _PC_MIN[1])
    lz = _bf(izf * _VOX_SZ[2] + _PC_MIN[2])

    cam = [l2c[4 * i] * lx + l2c[4 * i + 1] * ly + l2c[4 * i + 2] * lz
           + l2c[4 * i + 3] for i in range(3)]
    cb = [_bf(c) for c in cam]
    p0 = c2i[0] * cb[0] + c2i[1] * cb[1] + c2i[2] * cb[2] + c2i[3]
    p1 = c2i[4] * cb[0] + c2i[5] * cb[1] + c2i[6] * cb[2] + c2i[7]
    p2 = c2i[8] * cb[0] + c2i[9] * cb[1] + c2i[10] * cb[2] + c2i[11]

    u = p0 / p2
    v = p1 / p2
    depth = p2 - c2i23
    arg = 1.0 + 8.0 * (depth - DEPTH_MIN) / BIN_SIZE
    dbin = jnp.where(arg >= 0.0,
                     -0.5 + 0.5 * jnp.sqrt(jnp.maximum(arg, 0.0)),
                     jnp.nan)

    gu = u / nw * 2.0 - 1.0
    gv = v / nh * 2.0 - 1.0
    gd = dbin / jnp.float32(NUM_BINS - 1) * 2.0 - 1.0
    gu = jnp.where(jnp.isfinite(gu), gu, OOB)
    gv = jnp.where(jnp.isfinite(gv), gv, OOB)
    gd = jnp.where(jnp.isfinite(gd), gd, OOB)

    gx = (gu + 1.0) * 0.5 * (W_FEAT - 1)
    gy = (gv + 1.0) * 0.5 * (H_FEAT - 1)
    gz = (gd + 1.0) * 0.5 * (D_BINS - 1)

    x0 = jnp.floor(gx)
    y0 = jnp.floor(gy)
    z0 = jnp.floor(gz)

    def axis_terms(g, g0, hi):
        res = []
        for d in (0.0, 1.0):
            gi = g0 + d
            w_ = 1.0 - jnp.abs(g - gi)
            valid = ((gi >= 0.0) & (gi <= hi)).astype(jnp.float32)
            cl = jnp.clip(gi, 0.0, hi).astype(jnp.int32)
            res.append((w_ * valid, cl))
        return res

    ax = axis_terms(gx, x0, W_FEAT - 1)
    ay = axis_terms(gy, y0, H_FEAT - 1)
    az = axis_terms(gz, z0, D_BINS - 1)

    k = 0
    for dz in (0, 1):
        wz_, zc = az[dz]
        for dy in (0, 1):
            wy_, yc = ay[dy]
            for dx in (0, 1):
                wx_, xc = ax[dx]
                w_ref[k] = (wx_ * wy_) * wz_
                idx_ref[k] = (zc * H_FEAT + yc) * W_FEAT + xc
                k += 1


def _make_grid(params):
    return pl.pallas_call(
        _grid_body,
        grid=(GZ,),
        in_specs=[pl.BlockSpec(memory_space=pltpu.SMEM)],
        out_specs=[
            pl.BlockSpec((8, 1, GY, GX), lambda z: (0, z, 0, 0)),
            pl.BlockSpec((8, 1, GY, GX), lambda z: (0, z, 0, 0)),
        ],
        out_shape=[
            jax.ShapeDtypeStruct((8, GZ, GY, GX), jnp.int32),
            jax.ShapeDtypeStruct((8, GZ, GY, GX), jnp.float32),
        ],
    )(params)


def _sc_gather_body(idx_hbm, w_hbm, table_hbm, out_hbm,
                    idx_v, w_v, rows_v, outt_v, sem):
    cid = lax.axis_index("c")
    sid = lax.axis_index("s")
    wid = sid * 2 + cid
    base_w = wid * PER_W

    row_iv = [(lax.iota(jnp.int32, 16) + 16 * b) * NBLK for b in range(4)]

    def chunk_body(ch, carry):
        base = base_w + ch * NBLK
        pltpu.sync_copy(idx_hbm.at[:, pl.ds(base, NBLK)], idx_v)
        pltpu.sync_copy(w_hbm.at[:, pl.ds(base, NBLK)], w_v)
        cps = [pltpu.async_copy(table_hbm.at[idx_v.at[kk]], rows_v.at[kk], sem)
               for kk in range(8)]
        for cp in cps:
            cp.wait()

        def grp_body(g, vcarry):
            vb = g * 16
            wvecs = [w_v[kk, pl.ds(vb, 16)] for kk in range(8)]
            for j in range(16):
                ws = [wvecs[kk][j] for kk in range(8)]
                vi = vb + j
                for b in range(4):
                    sl = pl.ds(16 * b, 16)
                    acc = ws[0] * rows_v[0, vi, sl]
                    for kk in range(1, 8):
                        acc = acc + ws[kk] * rows_v[kk, vi, sl]
                    plsc.store_scatter(outt_v, [row_iv[b] + vi], acc)
            return vcarry

        lax.fori_loop(0, NBLK // 16, grp_body, 0)
        ocps = [pltpu.async_copy(outt_v.at[pl.ds(c * NBLK, NBLK)],
                                 out_hbm.at[c, pl.ds(base, NBLK)], sem)
                for c in range(C)]
        for cp in ocps:
            cp.wait()
        return carry

    lax.fori_loop(0, NCHUNK, chunk_body, 0)


def _sc_gather(idx8, w8, table):
    mesh = plsc.VectorSubcoreMesh(core_axis_name="c", subcore_axis_name="s")
    fn = functools.partial(
        pl.kernel,
        mesh=mesh,
        out_type=jax.ShapeDtypeStruct((C, N_VOX), jnp.float32),
        scratch_types=[
            pltpu.VMEM((8, NBLK), jnp.int32),
            pltpu.VMEM((8, NBLK), jnp.float32),
            pltpu.VMEM((8, NBLK, C), jnp.float32),
            pltpu.VMEM((C * NBLK,), jnp.float32),
            pltpu.SemaphoreType.DMA,
        ],
        compiler_params=pltpu.CompilerParams(needs_layout_passes=False,
                                             use_tc_tiling_on_sc=False),
    )(_sc_gather_body)
    return fn(idx8, w8, table)


def kernel(frustum_features, lidar_to_cam, cam_to_img, image_shape):
    feat2d = frustum_features.reshape(C, DHW)
    table = _relayout_features(feat2d)

    l2cb = lidar_to_cam[0, :3].astype(jnp.bfloat16).astype(jnp.float32)
    c2ib = cam_to_img[0].astype(jnp.bfloat16).astype(jnp.float32)
    img = jnp.max(image_shape, axis=0).astype(jnp.float32)
    params = jnp.concatenate([
        l2cb.reshape(12),
        c2ib.reshape(12),
        jnp.stack([cam_to_img[0, 2, 3], img[1] - 1.0, img[0] - 1.0,
                   jnp.float32(0.0)]),
    ])

    idx8, w8 = _make_grid(params)
    idx8 = idx8.reshape(8, N_VOX)
    w8 = w8.reshape(8, N_VOX)

    out = _sc_gather(idx8, w8, table)
    return out.reshape(1, C, GZ, GY, GX)
